# baseline (speedup 1.0000x reference)
import functools

import jax
import jax.numpy as jnp
from jax import lax
from jax.experimental import pallas as pl
from jax.experimental.pallas import tpu as pltpu
from jax.experimental.pallas import tpu_sc as plsc

N_PTS = 2_000_000
GRID = 128
CELLS = GRID - 1
TBL = CELLS * CELLS * CELLS
SX = CELLS * CELLS
SY = CELLS

NW = 32
C = 3200
NCH = N_PTS // C

_mesh = plsc.VectorSubcoreMesh(
    core_axis_name="c", subcore_axis_name="s", num_cores=2, num_subcores=16
)


def _tec_body(r_hbm, phi_hbm, tab_hbm, out_hbm,
              rv, phiv, fxv, fyv, fzv, idxv, pvals, outv, gsem):
    wid = lax.axis_index("s") * 2 + lax.axis_index("c")
    nchunks = jnp.where(wid <= 16, NCH // NW + 1, NCH // NW)
    iota = lax.iota(jnp.int32, 16)
    iota3 = iota * 3

    def do_chunk(i, carry):
        chunk = wid + i * NW
        base = chunk * C
        pltpu.sync_copy(r_hbm.at[pl.ds(base * 3, C * 3)], rv)
        pltpu.sync_copy(phi_hbm.at[pl.ds(base, C)], phiv)

        def idx_step(j, c2):
            o = j * 48
            xv = plsc.load_gather(rv, [iota3 + o])
            yv = plsc.load_gather(rv, [iota3 + (o + 1)])
            zv = plsc.load_gather(rv, [iota3 + (o + 2)])
            tx = (xv + 1.0) * 63.5
            ty = (yv + 1.0) * 63.5
            tz = (zv + 1.0) * 63.5
            ix = jnp.clip(tx.astype(jnp.int32), 0, GRID - 2)
            iy = jnp.clip(ty.astype(jnp.int32), 0, GRID - 2)
            iz = jnp.clip(tz.astype(jnp.int32), 0, GRID - 2)
            fxv[pl.ds(j * 16, 16)] = jnp.clip(tx - ix.astype(jnp.float32), 0.0, 1.0)
            fyv[pl.ds(j * 16, 16)] = jnp.clip(ty - iy.astype(jnp.float32), 0.0, 1.0)
            fzv[pl.ds(j * 16, 16)] = jnp.clip(tz - iz.astype(jnp.float32), 0.0, 1.0)
            pv = phiv[pl.ds(j * 16, 16)]
            idxv[pl.ds(j * 16, 16)] = (
                ix * SX + iy * SY + iz + jnp.where(pv < 0.0, TBL, 0))
            return c2

        lax.fori_loop(0, C // 16, idx_step, 0)
        pltpu.async_copy(tab_hbm.at[idxv], pvals, gsem)
        pltpu.make_async_copy(tab_hbm.at[idxv], pvals, gsem).wait()

        def comb_step(j, c2):
            rows = iota + j * 16
            v = [plsc.load_gather(pvals, [rows, jnp.full((16,), c, jnp.int32)])
                 for c in range(8)]
            fx = fxv[pl.ds(j * 16, 16)]
            fy = fyv[pl.ds(j * 16, 16)]
            fz = fzv[pl.ds(j * 16, 16)]
            c00 = v[0] * (1.0 - fx) + v[4] * fx
            c01 = v[1] * (1.0 - fx) + v[5] * fx
            c10 = v[2] * (1.0 - fx) + v[6] * fx
            c11 = v[3] * (1.0 - fx) + v[7] * fx
            c0 = c00 * (1.0 - fy) + c10 * fy
            c1 = c01 * (1.0 - fy) + c11 * fy
            outv[pl.ds(j * 16, 16)] = c0 * (1.0 - fz) + c1 * fz
            return c2

        lax.fori_loop(0, C // 16, comb_step, 0)
        pltpu.sync_copy(outv, out_hbm.at[pl.ds(base, C)])
        return carry

    lax.fori_loop(0, nchunks, do_chunk, 0)


_interp = functools.partial(
    pl.kernel,
    out_type=jax.ShapeDtypeStruct((N_PTS,), jnp.float32),
    mesh=_mesh,
    compiler_params=pltpu.CompilerParams(
        needs_layout_passes=False,
        use_tc_tiling_on_sc=False,
    ),
    scratch_types=[
        pltpu.VMEM((3 * C,), jnp.float32),
        pltpu.VMEM((C,), jnp.float32),
        pltpu.VMEM((C,), jnp.float32),
        pltpu.VMEM((C,), jnp.float32),
        pltpu.VMEM((C,), jnp.float32),
        pltpu.VMEM((C,), jnp.int32),
        pltpu.VMEM((C, 8), jnp.float32),
        pltpu.VMEM((C,), jnp.float32),
        pltpu.SemaphoreType.DMA,
    ],
)(_tec_body)


def _pack_corners(t):
    cs = [t[dx:dx + CELLS, dy:dy + CELLS, dz:dz + CELLS]
          for dx in (0, 1) for dy in (0, 1) for dz in (0, 1)]
    return jnp.stack(cs, axis=-1).reshape(TBL, 8)


def kernel(r, phi_r, trainables_m, trainables_p):
    tab = jnp.concatenate(
        [_pack_corners(trainables_p), _pack_corners(trainables_m)])
    return _interp(r.reshape(-1), phi_r, tab)


# speedup vs baseline: 15.4755x; 15.4755x over previous
import functools

import jax
import jax.numpy as jnp
from jax import lax
from jax.experimental import pallas as pl
from jax.experimental.pallas import tpu as pltpu
from jax.experimental.pallas import tpu_sc as plsc

N_PTS = 2_000_000
GRID = 128
CELLS = GRID - 1
TBL = CELLS * CELLS * CELLS
SXR = CELLS * CELLS
SYR = CELLS
PLANE = GRID * GRID

NW = 32
C = 3200
NCH = N_PTS // C

_mesh = plsc.VectorSubcoreMesh(
    core_axis_name="c", subcore_axis_name="s", num_cores=2, num_subcores=16
)
_params = pltpu.CompilerParams(
    needs_layout_passes=False,
    use_tc_tiling_on_sc=False,
)


def _build_body(tp_hbm, tm_hbm, tab_hbm, plA, plB, sbuf, gsem):
    wid = lax.axis_index("s") * 2 + lax.axis_index("c")
    nplanes = jnp.where(wid <= 30, 4, 3)
    iota = lax.iota(jnp.int32, 16)

    def build_one(t_hbm, tbl_off):
        def plane_body(k, carry):
            px = wid + k * NW
            pltpu.sync_copy(t_hbm.at[pl.ds(px * PLANE, PLANE)],
                            plA.at[pl.ds(0, PLANE)])
            pltpu.sync_copy(t_hbm.at[pl.ds((px + 1) * PLANE, PLANE)],
                            plB.at[pl.ds(0, PLANE)])

            def y_body(y, c2):
                slot = y & 3
                for j in range(8):
                    z0 = j * 16
                    mask = (iota + z0) < CELLS
                    rows = slot * CELLS + iota + z0
                    for c in range(8):
                        dx, dy, dz = (c >> 2) & 1, (c >> 1) & 1, c & 1
                        src = plB if dx else plA
                        vec = src[pl.ds((y + dy) * GRID + z0 + dz, 16)]
                        plsc.store_scatter(
                            sbuf, [rows, jnp.full((16,), c, jnp.int32)],
                            vec, mask=mask)
                stripe = tbl_off + px * SXR + y * SYR
                pltpu.async_copy(sbuf.at[pl.ds(slot * CELLS, CELLS)],
                                 tab_hbm.at[pl.ds(stripe, CELLS)], gsem)

                @pl.when(y >= 3)
                def _drain():
                    oy = y - 3
                    ostripe = tbl_off + px * SXR + oy * SYR
                    pltpu.make_async_copy(
                        sbuf.at[pl.ds((oy & 3) * CELLS, CELLS)],
                        tab_hbm.at[pl.ds(ostripe, CELLS)], gsem).wait()
                return c2

            lax.fori_loop(0, CELLS, y_body, 0)
            for oy in (CELLS - 3, CELLS - 2, CELLS - 1):
                ostripe = tbl_off + px * SXR + oy * SYR
                pltpu.make_async_copy(
                    sbuf.at[pl.ds((oy & 3) * CELLS, CELLS)],
                    tab_hbm.at[pl.ds(ostripe, CELLS)], gsem).wait()
            return carry

        lax.fori_loop(0, nplanes, plane_body, 0)

    build_one(tp_hbm, 0)
    build_one(tm_hbm, TBL)


_build = functools.partial(
    pl.kernel,
    out_type=jax.ShapeDtypeStruct((2 * TBL, 8), jnp.float32),
    mesh=_mesh,
    compiler_params=_params,
    scratch_types=[
        pltpu.VMEM((PLANE + 16,), jnp.float32),
        pltpu.VMEM((PLANE + 16,), jnp.float32),
        pltpu.VMEM((4 * CELLS, 8), jnp.float32),
        pltpu.SemaphoreType.DMA,
    ],
)(_build_body)


def _tec_body(rx_hbm, ry_hbm, rz_hbm, phi_hbm, tab_hbm, out_hbm,
              rxv, ryv, rzv, phiv, fxv, fyv, fzv, idxv, pvals, outv, gsem):
    wid = lax.axis_index("s") * 2 + lax.axis_index("c")
    nchunks = jnp.where(wid <= 16, NCH // NW + 1, NCH // NW)
    iota = lax.iota(jnp.int32, 16)

    def do_chunk(i, carry):
        chunk = wid + i * NW
        base = chunk * C
        pltpu.sync_copy(rx_hbm.at[pl.ds(base, C)], rxv)
        pltpu.sync_copy(ry_hbm.at[pl.ds(base, C)], ryv)
        pltpu.sync_copy(rz_hbm.at[pl.ds(base, C)], rzv)
        pltpu.sync_copy(phi_hbm.at[pl.ds(base, C)], phiv)

        def idx_step(j, c2):
            xv = rxv[pl.ds(j * 16, 16)]
            yv = ryv[pl.ds(j * 16, 16)]
            zv = rzv[pl.ds(j * 16, 16)]
            tx = (xv + 1.0) * 63.5
            ty = (yv + 1.0) * 63.5
            tz = (zv + 1.0) * 63.5
            ix = jnp.clip(tx.astype(jnp.int32), 0, GRID - 2)
            iy = jnp.clip(ty.astype(jnp.int32), 0, GRID - 2)
            iz = jnp.clip(tz.astype(jnp.int32), 0, GRID - 2)
            fxv[pl.ds(j * 16, 16)] = jnp.clip(tx - ix.astype(jnp.float32), 0.0, 1.0)
            fyv[pl.ds(j * 16, 16)] = jnp.clip(ty - iy.astype(jnp.float32), 0.0, 1.0)
            fzv[pl.ds(j * 16, 16)] = jnp.clip(tz - iz.astype(jnp.float32), 0.0, 1.0)
            pv = phiv[pl.ds(j * 16, 16)]
            idxv[pl.ds(j * 16, 16)] = (
                ix * SXR + iy * SYR + iz + jnp.where(pv < 0.0, TBL, 0))
            return c2

        lax.fori_loop(0, C // 16, idx_step, 0)
        pltpu.async_copy(tab_hbm.at[idxv], pvals, gsem)
        pltpu.make_async_copy(tab_hbm.at[idxv], pvals, gsem).wait()

        def comb_step(j, c2):
            rows = iota + j * 16
            v = [plsc.load_gather(pvals, [rows, jnp.full((16,), c, jnp.int32)])
                 for c in range(8)]
            fx = fxv[pl.ds(j * 16, 16)]
            fy = fyv[pl.ds(j * 16, 16)]
            fz = fzv[pl.ds(j * 16, 16)]
            c00 = v[0] * (1.0 - fx) + v[4] * fx
            c01 = v[1] * (1.0 - fx) + v[5] * fx
            c10 = v[2] * (1.0 - fx) + v[6] * fx
            c11 = v[3] * (1.0 - fx) + v[7] * fx
            c0 = c00 * (1.0 - fy) + c10 * fy
            c1 = c01 * (1.0 - fy) + c11 * fy
            outv[pl.ds(j * 16, 16)] = c0 * (1.0 - fz) + c1 * fz
            return c2

        lax.fori_loop(0, C // 16, comb_step, 0)
        pltpu.sync_copy(outv, out_hbm.at[pl.ds(base, C)])
        return carry

    lax.fori_loop(0, nchunks, do_chunk, 0)


_interp = functools.partial(
    pl.kernel,
    out_type=jax.ShapeDtypeStruct((N_PTS,), jnp.float32),
    mesh=_mesh,
    compiler_params=_params,
    scratch_types=[
        pltpu.VMEM((C,), jnp.float32),
        pltpu.VMEM((C,), jnp.float32),
        pltpu.VMEM((C,), jnp.float32),
        pltpu.VMEM((C,), jnp.float32),
        pltpu.VMEM((C,), jnp.float32),
        pltpu.VMEM((C,), jnp.float32),
        pltpu.VMEM((C,), jnp.float32),
        pltpu.VMEM((C,), jnp.int32),
        pltpu.VMEM((C, 8), jnp.float32),
        pltpu.VMEM((C,), jnp.float32),
        pltpu.SemaphoreType.DMA,
    ],
)(_tec_body)


def kernel(r, phi_r, trainables_m, trainables_p):
    rx, ry, rz = r[:, 0], r[:, 1], r[:, 2]
    tab = _build(trainables_p.reshape(-1), trainables_m.reshape(-1))
    return _interp(rx, ry, rz, phi_r, tab)


# speedup vs baseline: 16.5192x; 1.0674x over previous
import functools

import jax
import jax.numpy as jnp
from jax import lax
from jax.experimental import pallas as pl
from jax.experimental.pallas import tpu as pltpu
from jax.experimental.pallas import tpu_sc as plsc

N_PTS = 2_000_000
GRID = 128
CELLS = GRID - 1
TBL = CELLS * CELLS * CELLS
SXR = CELLS * CELLS
SYR = CELLS
PLANE = GRID * GRID

NW = 32
C = 3200
NCH = N_PTS // C

_mesh = plsc.VectorSubcoreMesh(
    core_axis_name="c", subcore_axis_name="s", num_cores=2, num_subcores=16
)
_params = pltpu.CompilerParams(
    needs_layout_passes=False,
    use_tc_tiling_on_sc=False,
)


def _build_body(tp_hbm, tm_hbm, tab_hbm, plA, plB, sbuf, gsem):
    wid = lax.axis_index("s") * 2 + lax.axis_index("c")
    nplanes = jnp.where(wid <= 30, 4, 3)
    iota = lax.iota(jnp.int32, 16)

    def build_one(t_hbm, tbl_off):
        def plane_body(k, carry):
            px = wid + k * NW
            pltpu.sync_copy(t_hbm.at[pl.ds(px * PLANE, PLANE)],
                            plA.at[pl.ds(0, PLANE)])
            pltpu.sync_copy(t_hbm.at[pl.ds((px + 1) * PLANE, PLANE)],
                            plB.at[pl.ds(0, PLANE)])

            def y_body(y, c2):
                slot = y & 3
                for j in range(8):
                    z0 = j * 16
                    mask = (iota + z0) < CELLS
                    rows = slot * CELLS + iota + z0
                    for c in range(8):
                        dx, dy, dz = (c >> 2) & 1, (c >> 1) & 1, c & 1
                        src = plB if dx else plA
                        vec = src[pl.ds((y + dy) * GRID + z0 + dz, 16)]
                        plsc.store_scatter(
                            sbuf, [rows, jnp.full((16,), c, jnp.int32)],
                            vec, mask=mask)
                stripe = tbl_off + px * SXR + y * SYR
                pltpu.async_copy(sbuf.at[pl.ds(slot * CELLS, CELLS)],
                                 tab_hbm.at[pl.ds(stripe, CELLS)], gsem)

                @pl.when(y >= 3)
                def _drain():
                    oy = y - 3
                    ostripe = tbl_off + px * SXR + oy * SYR
                    pltpu.make_async_copy(
                        sbuf.at[pl.ds((oy & 3) * CELLS, CELLS)],
                        tab_hbm.at[pl.ds(ostripe, CELLS)], gsem).wait()
                return c2

            lax.fori_loop(0, CELLS, y_body, 0)
            for oy in (CELLS - 3, CELLS - 2, CELLS - 1):
                ostripe = tbl_off + px * SXR + oy * SYR
                pltpu.make_async_copy(
                    sbuf.at[pl.ds((oy & 3) * CELLS, CELLS)],
                    tab_hbm.at[pl.ds(ostripe, CELLS)], gsem).wait()
            return carry

        lax.fori_loop(0, nplanes, plane_body, 0)

    build_one(tp_hbm, 0)
    build_one(tm_hbm, TBL)


_build = functools.partial(
    pl.kernel,
    out_type=jax.ShapeDtypeStruct((2 * TBL, 8), jnp.float32),
    mesh=_mesh,
    compiler_params=_params,
    scratch_types=[
        pltpu.VMEM((PLANE + 16,), jnp.float32),
        pltpu.VMEM((PLANE + 16,), jnp.float32),
        pltpu.VMEM((4 * CELLS, 8), jnp.float32),
        pltpu.SemaphoreType.DMA,
    ],
)(_build_body)


def _tec_body(rx_hbm, ry_hbm, rz_hbm, phi_hbm, tab_hbm, out_hbm,
              rxv, ryv, rzv, phiv, fxv, fyv, fzv, idxv, pvals, outv, gsem):
    wid = lax.axis_index("s") * 2 + lax.axis_index("c")
    nchunks = jnp.where(wid <= 16, NCH // NW + 1, NCH // NW)
    iota = lax.iota(jnp.int32, 16)

    def do_iter(i, carry):
        s = i & 1

        @pl.when(i < nchunks)
        def _produce():
            chunk = wid + i * NW
            base = chunk * C
            pltpu.sync_copy(rx_hbm.at[pl.ds(base, C)], rxv.at[s])
            pltpu.sync_copy(ry_hbm.at[pl.ds(base, C)], ryv.at[s])
            pltpu.sync_copy(rz_hbm.at[pl.ds(base, C)], rzv.at[s])
            pltpu.sync_copy(phi_hbm.at[pl.ds(base, C)], phiv.at[s])

            def idx_step(j, c2):
                xv = rxv[s, pl.ds(j * 16, 16)]
                yv = ryv[s, pl.ds(j * 16, 16)]
                zv = rzv[s, pl.ds(j * 16, 16)]
                tx = (xv + 1.0) * 63.5
                ty = (yv + 1.0) * 63.5
                tz = (zv + 1.0) * 63.5
                ix = jnp.clip(tx.astype(jnp.int32), 0, GRID - 2)
                iy = jnp.clip(ty.astype(jnp.int32), 0, GRID - 2)
                iz = jnp.clip(tz.astype(jnp.int32), 0, GRID - 2)
                fxv[s, pl.ds(j * 16, 16)] = jnp.clip(tx - ix.astype(jnp.float32), 0.0, 1.0)
                fyv[s, pl.ds(j * 16, 16)] = jnp.clip(ty - iy.astype(jnp.float32), 0.0, 1.0)
                fzv[s, pl.ds(j * 16, 16)] = jnp.clip(tz - iz.astype(jnp.float32), 0.0, 1.0)
                pv = phiv[s, pl.ds(j * 16, 16)]
                idxv[s, pl.ds(j * 16, 16)] = (
                    ix * SXR + iy * SYR + iz + jnp.where(pv < 0.0, TBL, 0))
                return c2

            lax.fori_loop(0, C // 16, idx_step, 0)
            pltpu.async_copy(tab_hbm.at[idxv.at[s]], pvals.at[s], gsem.at[s])

        @pl.when(i > 0)
        def _consume():
            ps = 1 - s
            pbase = (wid + (i - 1) * NW) * C
            pltpu.make_async_copy(tab_hbm.at[idxv.at[ps]], pvals.at[ps],
                                  gsem.at[ps]).wait()

            def comb_step(j, c2):
                rows = iota + j * 16
                v = [plsc.load_gather(pvals.at[ps],
                                      [rows, jnp.full((16,), c, jnp.int32)])
                     for c in range(8)]
                fx = fxv[ps, pl.ds(j * 16, 16)]
                fy = fyv[ps, pl.ds(j * 16, 16)]
                fz = fzv[ps, pl.ds(j * 16, 16)]
                c00 = v[0] * (1.0 - fx) + v[4] * fx
                c01 = v[1] * (1.0 - fx) + v[5] * fx
                c10 = v[2] * (1.0 - fx) + v[6] * fx
                c11 = v[3] * (1.0 - fx) + v[7] * fx
                c0 = c00 * (1.0 - fy) + c10 * fy
                c1 = c01 * (1.0 - fy) + c11 * fy
                outv[ps, pl.ds(j * 16, 16)] = c0 * (1.0 - fz) + c1 * fz
                return c2

            lax.fori_loop(0, C // 16, comb_step, 0)
            pltpu.sync_copy(outv.at[ps], out_hbm.at[pl.ds(pbase, C)])

        return carry

    lax.fori_loop(0, nchunks + 1, do_iter, 0)


_interp = functools.partial(
    pl.kernel,
    out_type=jax.ShapeDtypeStruct((N_PTS,), jnp.float32),
    mesh=_mesh,
    compiler_params=_params,
    scratch_types=[
        pltpu.VMEM((2, C), jnp.float32),
        pltpu.VMEM((2, C), jnp.float32),
        pltpu.VMEM((2, C), jnp.float32),
        pltpu.VMEM((2, C), jnp.float32),
        pltpu.VMEM((2, C), jnp.float32),
        pltpu.VMEM((2, C), jnp.float32),
        pltpu.VMEM((2, C), jnp.float32),
        pltpu.VMEM((2, C), jnp.int32),
        pltpu.VMEM((2, C, 8), jnp.float32),
        pltpu.VMEM((2, C), jnp.float32),
        pltpu.SemaphoreType.DMA((2,)),
    ],
)(_tec_body)


def kernel(r, phi_r, trainables_m, trainables_p):
    rx, ry, rz = r[:, 0], r[:, 1], r[:, 2]
    tab = _build(trainables_p.reshape(-1), trainables_m.reshape(-1))
    return _interp(rx, ry, rz, phi_r, tab)
